# baseline (speedup 1.0000x reference)
import functools

import jax
import jax.numpy as jnp
from jax import lax
from jax.experimental import pallas as pl
from jax.experimental.pallas import tpu as pltpu
from jax.experimental.pallas import tpu_sc as plsc

_NUM_CORES = 2
_NUM_SUBCORES = 16
_NUM_WORKERS = _NUM_CORES * _NUM_SUBCORES
_LANES = 16
_SC_SHARE = 409600
_TC_BLOCK = 286720


def _sc_make(n_sc):
    per_worker = n_sc // _NUM_WORKERS
    mesh = plsc.VectorSubcoreMesh(
        core_axis_name="c",
        subcore_axis_name="s",
        num_cores=_NUM_CORES,
        num_subcores=_NUM_SUBCORES,
    )

    @functools.partial(
        pl.kernel,
        out_type=jax.ShapeDtypeStruct((n_sc,), jnp.float32),
        mesh=mesh,
        scratch_types=[
            pltpu.VMEM((_LANES,), jnp.float32),
            pltpu.VMEM((per_worker,), jnp.int32),
            pltpu.VMEM((per_worker,), jnp.float32),
        ],
        compiler_params=pltpu.CompilerParams(
            needs_layout_passes=False, skip_device_barrier=True
        ),
    )
    def sc_kernel(idx_hbm, data_hbm, out_hbm, table_v, idx_v, out_v):
        wid = lax.axis_index("s") * _NUM_CORES + lax.axis_index("c")
        base = wid * per_worker
        pltpu.sync_copy(idx_hbm.at[pl.ds(base, per_worker)], idx_v)
        pltpu.sync_copy(data_hbm, table_v.at[pl.ds(0, 8)])

        @plsc.parallel_loop(0, per_worker // _LANES, unroll=16)
        def _(i):
            iv = idx_v[pl.ds(i * _LANES, _LANES)]
            out_v[pl.ds(i * _LANES, _LANES)] = plsc.load_gather(table_v, [iv])

        pltpu.sync_copy(out_v, out_hbm.at[pl.ds(base, per_worker)])

    return sc_kernel


def _tc_body(table_ref, idx_ref, out_ref):
    idx = idx_ref[...]
    b0 = (idx & 1) != 0
    b1 = (idx & 2) != 0
    b2 = (idx & 4) != 0
    t = [table_ref[k] for k in range(8)]
    s01 = jnp.where(b0, t[1], t[0])
    s23 = jnp.where(b0, t[3], t[2])
    s45 = jnp.where(b0, t[5], t[4])
    s67 = jnp.where(b0, t[7], t[6])
    s0123 = jnp.where(b1, s23, s01)
    s4567 = jnp.where(b1, s67, s45)
    out_ref[...] = jnp.where(b2, s4567, s0123)


@jax.jit
def _hybrid_gather(indices, data):
    n = indices.shape[0]
    n_tc = n - _SC_SHARE
    out_sc = _sc_make(_SC_SHARE)(indices[:_SC_SHARE], data)
    out_tc = pl.pallas_call(
        _tc_body,
        grid=(n_tc // _TC_BLOCK,),
        in_specs=[
            pl.BlockSpec(memory_space=pltpu.SMEM),
            pl.BlockSpec((_TC_BLOCK,), lambda i: (i,)),
        ],
        out_specs=pl.BlockSpec((_TC_BLOCK,), lambda i: (i,)),
        out_shape=jax.ShapeDtypeStruct((n_tc,), jnp.float32),
    )(data, indices[_SC_SHARE:])
    return jnp.concatenate([out_sc, out_tc])


def kernel(indices, data):
    idx = indices.astype(jnp.int32)
    return _hybrid_gather(idx, data.astype(jnp.float32))


# speedup vs baseline: 1.6472x; 1.6472x over previous
import functools

import jax
import jax.numpy as jnp
from jax import lax
from jax.experimental import pallas as pl
from jax.experimental.pallas import tpu as pltpu
from jax.experimental.pallas import tpu_sc as plsc

_NUM_CORES = 2
_NUM_SUBCORES = 16
_NUM_WORKERS = _NUM_CORES * _NUM_SUBCORES
_LANES = 16
_NBUF = 4


@functools.partial(jax.jit, static_argnames=("chunk",))
def _sc_gather(indices, data, chunk=12800):
    n = indices.shape[0]
    per_worker = n // _NUM_WORKERS
    num_chunks = per_worker // chunk
    mesh = plsc.VectorSubcoreMesh(
        core_axis_name="c",
        subcore_axis_name="s",
        num_cores=_NUM_CORES,
        num_subcores=_NUM_SUBCORES,
    )

    @functools.partial(
        pl.kernel,
        out_type=jax.ShapeDtypeStruct((n,), jnp.float32),
        mesh=mesh,
        scratch_types=[
            pltpu.VMEM((_LANES,), jnp.float32),
            [pltpu.VMEM((chunk,), jnp.int32) for _ in range(_NBUF)],
            [pltpu.VMEM((chunk,), jnp.float32) for _ in range(_NBUF)],
            [pltpu.SemaphoreType.DMA for _ in range(_NBUF)],
            [pltpu.SemaphoreType.DMA for _ in range(_NBUF)],
        ],
        compiler_params=pltpu.CompilerParams(
            needs_layout_passes=False, skip_device_barrier=True
        ),
    )
    def gather_kernel(
        idx_hbm, data_hbm, out_hbm, table_v, idx_bufs, out_bufs, sin, sout
    ):
        wid = lax.axis_index("s") * _NUM_CORES + lax.axis_index("c")
        base = wid * per_worker

        def in_copy(c, b):
            off = base + c * chunk
            return pltpu.make_async_copy(
                idx_hbm.at[pl.ds(off, chunk)], idx_bufs[b], sin[b]
            )

        def out_copy(c, b):
            off = base + c * chunk
            return pltpu.make_async_copy(
                out_bufs[b], out_hbm.at[pl.ds(off, chunk)], sout[b]
            )


        num_groups = num_chunks // _NBUF

        for b in range(_NBUF):
            in_copy(b, b).start()
        pltpu.sync_copy(data_hbm, table_v.at[pl.ds(0, 8)])

        def group_body(g, carry):
            for b in range(_NBUF):
                c = g * _NBUF + b
                in_copy(c, b).wait()

                @pl.when(g > 0)
                def _():
                    out_copy(c - _NBUF, b).wait()

                @plsc.parallel_loop(0, chunk // _LANES, unroll=16)
                def _(i, idx_v=idx_bufs[b], out_v=out_bufs[b]):
                    iv = idx_v[pl.ds(i * _LANES, _LANES)]
                    out_v[pl.ds(i * _LANES, _LANES)] = plsc.load_gather(
                        table_v, [iv]
                    )

                out_copy(c, b).start()

                @pl.when(g < num_groups - 1)
                def _():
                    in_copy(c + _NBUF, b).start()

            return carry

        lax.fori_loop(0, num_groups, group_body, 0)

        for b in range(_NBUF):
            out_copy((num_groups - 1) * _NBUF + b, b).wait()

    return gather_kernel(indices, data)


def kernel(indices, data):
    idx = indices.astype(jnp.int32)
    return _sc_gather(idx, data.astype(jnp.float32))


# speedup vs baseline: 1.6475x; 1.0002x over previous
import functools

import jax
import jax.numpy as jnp
from jax import lax
from jax.experimental import pallas as pl
from jax.experimental.pallas import tpu as pltpu
from jax.experimental.pallas import tpu_sc as plsc

_NUM_CORES = 2
_NUM_SUBCORES = 16
_NUM_WORKERS = _NUM_CORES * _NUM_SUBCORES
_LANES = 16
_NBUF = 8


@functools.partial(jax.jit, static_argnames=("chunk",))
def _sc_gather(indices, data, chunk=6400):
    n = indices.shape[0]
    per_worker = n // _NUM_WORKERS
    num_chunks = per_worker // chunk
    mesh = plsc.VectorSubcoreMesh(
        core_axis_name="c",
        subcore_axis_name="s",
        num_cores=_NUM_CORES,
        num_subcores=_NUM_SUBCORES,
    )

    @functools.partial(
        pl.kernel,
        out_type=jax.ShapeDtypeStruct((n,), jnp.float32),
        mesh=mesh,
        scratch_types=[
            pltpu.VMEM((_LANES,), jnp.float32),
            [pltpu.VMEM((chunk,), jnp.int32) for _ in range(_NBUF)],
            [pltpu.VMEM((chunk,), jnp.float32) for _ in range(_NBUF)],
            [pltpu.SemaphoreType.DMA for _ in range(_NBUF)],
            [pltpu.SemaphoreType.DMA for _ in range(_NBUF)],
        ],
        compiler_params=pltpu.CompilerParams(
            needs_layout_passes=False, skip_device_barrier=True
        ),
    )
    def gather_kernel(
        idx_hbm, data_hbm, out_hbm, table_v, idx_bufs, out_bufs, sin, sout
    ):
        wid = lax.axis_index("s") * _NUM_CORES + lax.axis_index("c")
        base = wid * per_worker

        def in_copy(c, b):
            off = base + c * chunk
            return pltpu.make_async_copy(
                idx_hbm.at[pl.ds(off, chunk)], idx_bufs[b], sin[b]
            )

        def out_copy(c, b):
            off = base + c * chunk
            return pltpu.make_async_copy(
                out_bufs[b], out_hbm.at[pl.ds(off, chunk)], sout[b]
            )


        num_groups = num_chunks // _NBUF

        for b in range(_NBUF):
            in_copy(b, b).start()
        pltpu.sync_copy(data_hbm, table_v.at[pl.ds(0, 8)])

        def group_body(g, carry):
            for b in range(_NBUF):
                c = g * _NBUF + b
                in_copy(c, b).wait()

                @pl.when(g > 0)
                def _():
                    out_copy(c - _NBUF, b).wait()

                @plsc.parallel_loop(0, chunk // _LANES, unroll=16)
                def _(i, idx_v=idx_bufs[b], out_v=out_bufs[b]):
                    iv = idx_v[pl.ds(i * _LANES, _LANES)]
                    out_v[pl.ds(i * _LANES, _LANES)] = plsc.load_gather(
                        table_v, [iv]
                    )

                out_copy(c, b).start()

                @pl.when(g < num_groups - 1)
                def _():
                    in_copy(c + _NBUF, b).start()

            return carry

        lax.fori_loop(0, num_groups, group_body, 0)

        for b in range(_NBUF):
            out_copy((num_groups - 1) * _NBUF + b, b).wait()

    return gather_kernel(indices, data)


def kernel(indices, data):
    idx = indices.astype(jnp.int32)
    return _sc_gather(idx, data.astype(jnp.float32))
